# baseline (speedup 1.0000x reference)
import functools

import jax
import jax.numpy as jnp
from jax import lax
from jax.experimental import pallas as pl
from jax.experimental.pallas import tpu as pltpu
from jax.experimental.pallas import tpu_sc as plsc

_NC = 2
_NS = 16
_L = 16
_NW = _NC * _NS

_C = 128
_KC = 80
_KCB = 8


def _edge_mm_body(ea_ref, we_ref, be_ref, out_ref):
    out_ref[...] = (
        jnp.dot(ea_ref[...], we_ref[...], preferred_element_type=jnp.float32)
        + be_ref[...]
    )


def _edge_project(edge_attr, We, be, e_rows):
    E, ED = edge_attr.shape
    D = We.shape[1]
    BE = 4096
    grid = e_rows // BE
    return pl.pallas_call(
        _edge_mm_body,
        grid=(grid,),
        in_specs=[
            pl.BlockSpec((BE, ED), lambda i: (i, 0)),
            pl.BlockSpec((ED, D), lambda i: (0, 0)),
            pl.BlockSpec((1, D), lambda i: (0, 0)),
        ],
        out_specs=pl.BlockSpec((BE, D), lambda i: (i, 0)),
        out_shape=jax.ShapeDtypeStruct((e_rows, D), jnp.float32),
    )(edge_attr, We, be.reshape(1, D))


def _make_sc_agg(N, NP, D, KC):
    RPT = NP // _NS
    KOUT = RPT // _C
    mesh = plsc.VectorSubcoreMesh(core_axis_name="c", subcore_axis_name="s",
                                  num_cores=_NC, num_subcores=_NS)

    @functools.partial(
        pl.kernel,
        mesh=mesh,
        out_type=jax.ShapeDtypeStruct((_NC * NP, D), jnp.float32),
        scratch_types=[
            pltpu.VMEM((_KCB, _C), jnp.int32),
            pltpu.VMEM((_KCB, _C), jnp.int32),
            pltpu.VMEM((_C, D), jnp.float32),
            pltpu.VMEM((_C, D), jnp.float32),
            pltpu.VMEM_SHARED((NP, D), jnp.float32),
            pltpu.SemaphoreType.DMA,
        ],
    )
    def sc_agg(x_hbm, src_hbm, dst_hbm, e_hbm, out_hbm,
               src_v, dst_v, xv, ev, agg, sem):
        c = lax.axis_index("c")
        s = lax.axis_index("s")
        wid = c * _NS + s

        def zrow(r, carry):
            for cc in range(D // _L):
                xv[r, pl.ds(cc * _L, _L)] = jnp.zeros((_L,), jnp.float32)
            return carry

        lax.fori_loop(0, _C, zrow, 0)
        for k in range(KOUT):
            pltpu.sync_copy(
                xv, agg.at[pl.ds(pl.multiple_of(s * RPT + k * _C, 8), _C)])
        plsc.subcore_barrier()

        def blk(jj, carry):
            ibase = pl.multiple_of(wid * KC + jj * _KCB, 8)
            pltpu.sync_copy(src_hbm.at[pl.ds(ibase, _KCB)], src_v)
            pltpu.sync_copy(dst_hbm.at[pl.ds(ibase, _KCB)], dst_v)

            def chunk(jb, carry1):
                erow = pl.multiple_of(
                    (wid * KC + jj * _KCB + jb) * _C, 8)
                pltpu.sync_copy(e_hbm.at[pl.ds(erow, _C)], ev)
                pltpu.async_copy(x_hbm.at[src_v.at[jb]], xv, sem).wait()

                def row(r, carry2):
                    for cc in range(D // _L):
                        sl = pl.ds(cc * _L, _L)
                        xv[r, sl] = jnp.maximum(xv[r, sl] + ev[r, sl], 0.0)
                    return carry2

                lax.fori_loop(0, _C, row, 0)
                pltpu.sync_copy(xv, agg.at[dst_v.at[jb]], add=True)
                return carry1

            lax.fori_loop(0, _KCB, chunk, 0)
            return carry

        lax.fori_loop(0, KC // _KCB, blk, 0)
        plsc.subcore_barrier()

        for k in range(KOUT):
            off = pl.multiple_of(s * RPT + k * _C, 8)
            pltpu.sync_copy(agg.at[pl.ds(off, _C)],
                            out_hbm.at[pl.ds(pl.multiple_of(c * NP, 8) + off,
                                             _C)])

    return sc_agg


def _mlp_body(x_ref, pa0_ref, pa1_ref, pb0_ref, pb1_ref, pc0_ref, pc1_ref,
              pd0_ref, pd1_ref, pe0_ref, pe1_ref, w1_ref, b1_ref,
              w2_ref, b2_ref, g_ref, bb_ref, o_ref):
    h = (x_ref[...] + pa0_ref[0] + pa1_ref[0] + pb0_ref[0] + pb1_ref[0]
         + pc0_ref[0] + pc1_ref[0] + pd0_ref[0] + pd1_ref[0]
         + pe0_ref[0] + pe1_ref[0])
    t = jnp.maximum(
        jnp.dot(h, w1_ref[...], preferred_element_type=jnp.float32)
        + b1_ref[...], 0.0)
    h2 = (jnp.dot(t, w2_ref[...], preferred_element_type=jnp.float32)
          + b2_ref[...])
    mu = jnp.mean(h2, axis=-1, keepdims=True)
    var = jnp.mean((h2 - mu) ** 2, axis=-1, keepdims=True)
    hn = (h2 - mu) * lax.rsqrt(var + 1e-5) * g_ref[...] + bb_ref[...]
    o_ref[...] = jnp.maximum(hn, 0.0)


def _node_update(x, pa3, pb3, pc3, pd3, pe3, W1, b1, W2, b2, gamma, beta):
    N, D = x.shape
    BN = 2000
    grid = N // BN
    full = lambda i: (0, 0)
    return pl.pallas_call(
        _mlp_body,
        grid=(grid,),
        in_specs=[
            pl.BlockSpec((BN, D), lambda i: (i, 0)),
            pl.BlockSpec((1, BN, D), lambda i: (0, i, 0)),
            pl.BlockSpec((1, BN, D), lambda i: (1, i, 0)),
            pl.BlockSpec((1, BN, D), lambda i: (0, i, 0)),
            pl.BlockSpec((1, BN, D), lambda i: (1, i, 0)),
            pl.BlockSpec((1, BN, D), lambda i: (0, i, 0)),
            pl.BlockSpec((1, BN, D), lambda i: (1, i, 0)),
            pl.BlockSpec((1, BN, D), lambda i: (0, i, 0)),
            pl.BlockSpec((1, BN, D), lambda i: (1, i, 0)),
            pl.BlockSpec((1, BN, D), lambda i: (0, i, 0)),
            pl.BlockSpec((1, BN, D), lambda i: (1, i, 0)),
            pl.BlockSpec((D, D), full),
            pl.BlockSpec((1, D), full),
            pl.BlockSpec((D, D), full),
            pl.BlockSpec((1, D), full),
            pl.BlockSpec((1, D), full),
            pl.BlockSpec((1, D), full),
        ],
        out_specs=pl.BlockSpec((BN, D), lambda i: (i, 0)),
        out_shape=jax.ShapeDtypeStruct((N, D), jnp.float32),
    )(x, pa3, pa3, pb3, pb3, pc3, pc3, pd3, pd3, pe3, pe3,
      W1, b1.reshape(1, D),
      W2, b2.reshape(1, D), gamma.reshape(1, D), beta.reshape(1, D))


def kernel(x, edge_index, edge_attr, We, be, W1, b1, W2, b2, gamma, beta):
    N, D = x.shape
    E = edge_attr.shape[0]
    ED = edge_attr.shape[1]
    NSL = 5
    EP = _NW * _KC * _C
    EPS = EP // NSL
    KCS = _KC // NSL
    NP = ((N // _NS) // _C + 1) * _C * _NS
    pad = EP - E

    pad_dst = N + jnp.arange(pad, dtype=jnp.int32) % (NP - N)
    src_p = jnp.concatenate([edge_index[0], jnp.zeros((pad,), jnp.int32)])
    dst_p = jnp.concatenate([edge_index[1], pad_dst])
    ea_p = jnp.concatenate([edge_attr, jnp.zeros((pad, ED), jnp.float32)])

    sc = _make_sc_agg(N, NP, D, KCS)
    parts = []
    for q in range(NSL):
        lo, hi = q * EPS, (q + 1) * EPS
        eq = _edge_project(ea_p[lo:hi], We, be, EPS)
        pq = sc(x, src_p[lo:hi].reshape(EPS // _C, _C),
                dst_p[lo:hi].reshape(EPS // _C, _C), eq)
        parts.append(pq.reshape(_NC, NP, D))
    return _node_update(x, parts[0], parts[1], parts[2], parts[3], parts[4],
                        W1, b1, W2, b2, gamma, beta)


# speedup vs baseline: 1.0100x; 1.0100x over previous
import functools

import jax
import jax.numpy as jnp
from jax import lax
from jax.experimental import pallas as pl
from jax.experimental.pallas import tpu as pltpu
from jax.experimental.pallas import tpu_sc as plsc

_NC = 2
_NS = 16
_L = 16
_NW = _NC * _NS

_C = 128
_KC = 80
_KCB = 8


def _edge_mm_body(ea_ref, we_ref, be_ref, out_ref):
    out_ref[...] = (
        jnp.dot(ea_ref[...], we_ref[...], preferred_element_type=jnp.float32)
        + be_ref[...]
    )


def _edge_project(edge_attr, We, be, e_rows):
    E, ED = edge_attr.shape
    D = We.shape[1]
    BE = 4096
    grid = e_rows // BE
    return pl.pallas_call(
        _edge_mm_body,
        grid=(grid,),
        in_specs=[
            pl.BlockSpec((BE, ED), lambda i: (i, 0)),
            pl.BlockSpec((ED, D), lambda i: (0, 0)),
            pl.BlockSpec((1, D), lambda i: (0, 0)),
        ],
        out_specs=pl.BlockSpec((BE, D), lambda i: (i, 0)),
        out_shape=jax.ShapeDtypeStruct((e_rows, D), jnp.float32),
    )(edge_attr, We, be.reshape(1, D))


def _make_sc_agg(N, NP, D, KC):
    RPT = NP // _NS
    KOUT = RPT // _C
    mesh = plsc.VectorSubcoreMesh(core_axis_name="c", subcore_axis_name="s",
                                  num_cores=_NC, num_subcores=_NS)

    @functools.partial(
        pl.kernel,
        mesh=mesh,
        out_type=jax.ShapeDtypeStruct((_NC * NP, D), jnp.float32),
        scratch_types=[
            pltpu.VMEM((_KCB, _C), jnp.int32),
            pltpu.VMEM((_KCB, _C), jnp.int32),
            pltpu.VMEM((_C, D), jnp.float32),
            pltpu.VMEM((_C, D), jnp.float32),
            pltpu.VMEM_SHARED((NP, D), jnp.float32),
            pltpu.SemaphoreType.DMA,
        ],
    )
    def sc_agg(x_hbm, src_hbm, dst_hbm, e_hbm, out_hbm,
               src_v, dst_v, xv, ev, agg, sem):
        c = lax.axis_index("c")
        s = lax.axis_index("s")
        wid = c * _NS + s

        def zrow(r, carry):
            for cc in range(D // _L):
                xv[r, pl.ds(cc * _L, _L)] = jnp.zeros((_L,), jnp.float32)
            return carry

        lax.fori_loop(0, _C, zrow, 0)
        for k in range(KOUT):
            pltpu.sync_copy(
                xv, agg.at[pl.ds(pl.multiple_of(s * RPT + k * _C, 8), _C)])
        plsc.subcore_barrier()

        def blk(jj, carry):
            ibase = pl.multiple_of(wid * KC + jj * _KCB, 8)
            pltpu.sync_copy(src_hbm.at[pl.ds(ibase, _KCB)], src_v)
            pltpu.sync_copy(dst_hbm.at[pl.ds(ibase, _KCB)], dst_v)

            def chunk(jb, carry1):
                erow = pl.multiple_of(
                    (wid * KC + jj * _KCB + jb) * _C, 8)
                pltpu.sync_copy(e_hbm.at[pl.ds(erow, _C)], ev)
                pltpu.async_copy(x_hbm.at[src_v.at[jb]], xv, sem).wait()

                def row(r, carry2):
                    for cc in range(D // _L):
                        sl = pl.ds(cc * _L, _L)
                        xv[r, sl] = jnp.maximum(xv[r, sl] + ev[r, sl], 0.0)
                    return carry2

                lax.fori_loop(0, _C, row, 0)
                pltpu.sync_copy(xv, agg.at[dst_v.at[jb]], add=True)
                return carry1

            lax.fori_loop(0, _KCB, chunk, 0)
            return carry

        lax.fori_loop(0, KC // _KCB, blk, 0)
        plsc.subcore_barrier()

        for k in range(KOUT):
            off = pl.multiple_of(s * RPT + k * _C, 8)
            pltpu.sync_copy(agg.at[pl.ds(off, _C)],
                            out_hbm.at[pl.ds(pl.multiple_of(c * NP, 8) + off,
                                             _C)])

    return sc_agg


def _mlp_body(x_ref, pa0_ref, pa1_ref, pb0_ref, pb1_ref, w1_ref, b1_ref,
              w2_ref, b2_ref, g_ref, bb_ref, o_ref):
    h = (x_ref[...] + pa0_ref[0] + pa1_ref[0] + pb0_ref[0] + pb1_ref[0])
    t = jnp.maximum(
        jnp.dot(h, w1_ref[...], preferred_element_type=jnp.float32)
        + b1_ref[...], 0.0)
    h2 = (jnp.dot(t, w2_ref[...], preferred_element_type=jnp.float32)
          + b2_ref[...])
    mu = jnp.mean(h2, axis=-1, keepdims=True)
    var = jnp.mean((h2 - mu) ** 2, axis=-1, keepdims=True)
    hn = (h2 - mu) * lax.rsqrt(var + 1e-5) * g_ref[...] + bb_ref[...]
    o_ref[...] = jnp.maximum(hn, 0.0)


def _node_update(x, pa3, pb3, W1, b1, W2, b2, gamma, beta):
    N, D = x.shape
    BN = 2000
    grid = N // BN
    full = lambda i: (0, 0)
    return pl.pallas_call(
        _mlp_body,
        grid=(grid,),
        in_specs=[
            pl.BlockSpec((BN, D), lambda i: (i, 0)),
            pl.BlockSpec((1, BN, D), lambda i: (0, i, 0)),
            pl.BlockSpec((1, BN, D), lambda i: (1, i, 0)),
            pl.BlockSpec((1, BN, D), lambda i: (0, i, 0)),
            pl.BlockSpec((1, BN, D), lambda i: (1, i, 0)),
            pl.BlockSpec((D, D), full),
            pl.BlockSpec((1, D), full),
            pl.BlockSpec((D, D), full),
            pl.BlockSpec((1, D), full),
            pl.BlockSpec((1, D), full),
            pl.BlockSpec((1, D), full),
        ],
        out_specs=pl.BlockSpec((BN, D), lambda i: (i, 0)),
        out_shape=jax.ShapeDtypeStruct((N, D), jnp.float32),
    )(x, pa3, pa3, pb3, pb3, W1, b1.reshape(1, D), W2, b2.reshape(1, D),
      gamma.reshape(1, D), beta.reshape(1, D))


def kernel(x, edge_index, edge_attr, We, be, W1, b1, W2, b2, gamma, beta):
    N, D = x.shape
    E = edge_attr.shape[0]
    EP = _NW * _KC * _C
    EPH = EP // 2
    KCH = _KC // 2
    NP = ((N // _NS) // _C + 1) * _C * _NS
    pad = EP - E

    eaA = edge_attr[:EPH]
    eaB = jnp.concatenate([edge_attr[EPH:],
                           jnp.zeros((pad, edge_attr.shape[1]), jnp.float32)])
    srcA = edge_index[0][:EPH].reshape(EPH // _C, _C)
    dstA = edge_index[1][:EPH].reshape(EPH // _C, _C)
    srcB = jnp.concatenate(
        [edge_index[0][EPH:], jnp.zeros((pad,), jnp.int32)]
    ).reshape(EPH // _C, _C)
    pad_dst = N + jnp.arange(pad, dtype=jnp.int32) % (NP - N)
    dstB = jnp.concatenate(
        [edge_index[1][EPH:], pad_dst]).reshape(EPH // _C, _C)

    sc = _make_sc_agg(N, NP, D, KCH)
    eA = _edge_project(eaA, We, be, EPH)
    pA = sc(x, srcA, dstA, eA)
    eB = _edge_project(eaB, We, be, EPH)
    pB = sc(x, srcB, dstB, eB)
    return _node_update(x, pA.reshape(_NC, NP, D), pB.reshape(_NC, NP, D),
                        W1, b1, W2, b2, gamma, beta)
